# baseline (speedup 1.0000x reference)
import functools

import jax
import jax.numpy as jnp
from jax import lax
from jax.experimental import pallas as pl
from jax.experimental.pallas import tpu as pltpu
from jax.experimental.pallas import tpu_sc as plsc

N_NODES = 10000
N_EDGES = 320000
D = 128

NC = 2
NS = 16
NW = NC * NS
EPT = N_EDGES // NW
CHUNK = 80
NCHUNK = EPT // CHUNK
NSTREAM = 2 * NCHUNK

WIN = N_NODES // 2
TRASH_GROUPS = 8
TRASH_PER_GROUP = 8
TRASH = TRASH_GROUPS * TRASH_PER_GROUP
ACC_ROWS = WIN + TRASH
WROWS_PER_TILE = 312
WTAIL = WIN - NS * WROWS_PER_TILE

ROW_BLK = 1000
N_BLKS = N_NODES // ROW_BLK


def _mm_body(x_ref, w2t_ref, feat_ref):
    feat_ref[...] = jnp.dot(x_ref[...], w2t_ref[...],
                            preferred_element_type=jnp.float32)


def _combine_body(x_ref, w1t_ref, p_ref, out_ref):
    s = jnp.dot(x_ref[...], w1t_ref[...], preferred_element_type=jnp.float32)
    out_ref[...] = jnp.maximum(s + p_ref[0] + p_ref[1], 0.0)


_sc_mesh = plsc.VectorSubcoreMesh(core_axis_name="c", subcore_axis_name="s")


@functools.partial(
    pl.kernel,
    out_type=jax.ShapeDtypeStruct((NC, N_NODES, D), jnp.float32),
    mesh=_sc_mesh,
    scratch_types=[
        pltpu.VMEM((NCHUNK, CHUNK), jnp.int32),
        pltpu.VMEM((NSTREAM, CHUNK), jnp.int32),
        pltpu.VMEM((CHUNK, D), jnp.float32),
        pltpu.VMEM((CHUNK, D), jnp.float32),
        pltpu.VMEM((CHUNK, D), jnp.float32),
        pltpu.VMEM((CHUNK, D), jnp.float32),
        pltpu.VMEM_SHARED((ACC_ROWS, D), jnp.float32),
        pltpu.SemaphoreType.DMA,
        pltpu.SemaphoreType.DMA,
        pltpu.SemaphoreType.DMA,
        pltpu.SemaphoreType.DMA,
    ],
)
def _aggregate(feats_hbm, col_hbm, rowcat_hbm, zeros_hbm, out_hbm,
               colv, rowv, buf0, buf1, buf2, buf3, acc,
               sem0, sem1, sem2, sem3):
    cid = lax.axis_index("c")
    sid = lax.axis_index("s")
    wid = cid * NS + sid
    bufs = (buf0, buf1, buf2, buf3)
    sems = (sem0, sem1, sem2, sem3)

    pltpu.sync_copy(col_hbm.at[wid], colv)
    pltpu.sync_copy(rowcat_hbm.at[wid], rowv)

    def gather(k, b):
        kk = jnp.minimum(k, NSTREAM - 1)
        src = kk - NCHUNK * (kk // NCHUNK)
        pltpu.async_copy(feats_hbm.at[colv.at[src]], bufs[b], sems[b])

    def wait_gather(b):
        pltpu.make_async_copy(feats_hbm.at[colv.at[0]], bufs[b],
                              sems[b]).wait()

    def zero_acc():
        zbase = sid * WROWS_PER_TILE
        for m in range(3):
            pltpu.sync_copy(zeros_hbm, acc.at[pl.ds(zbase + m * CHUNK, CHUNK)])
        pltpu.sync_copy(zeros_hbm.at[pl.ds(0, 72)],
                        acc.at[pl.ds(zbase + 3 * CHUNK, 72)])
        @pl.when(sid < TRASH_GROUPS)
        def _zero_trash():
            pltpu.sync_copy(zeros_hbm.at[pl.ds(0, TRASH_PER_GROUP)],
                            acc.at[pl.ds(WIN + sid * TRASH_PER_GROUP,
                                         TRASH_PER_GROUP)])

        @pl.when(sid == 0)
        def _zero_tail():
            pltpu.sync_copy(zeros_hbm.at[pl.ds(0, WTAIL)],
                            acc.at[pl.ds(NS * WROWS_PER_TILE, WTAIL)])

    def copy_out(sweep):
        obase = sweep * WIN
        pltpu.sync_copy(
            acc.at[pl.ds(sid * WROWS_PER_TILE, WROWS_PER_TILE)],
            out_hbm.at[cid, pl.ds(obase + sid * WROWS_PER_TILE,
                                  WROWS_PER_TILE)])

        @pl.when(sid == 0)
        def _copy_tail():
            pltpu.sync_copy(
                acc.at[pl.ds(NS * WROWS_PER_TILE, WTAIL)],
                out_hbm.at[cid, pl.ds(obase + NS * WROWS_PER_TILE, WTAIL)])

    for b in range(4):
        gather(jnp.int32(b), b)
    zero_acc()
    plsc.subcore_barrier()

    def quad0(j, carry):
        for b in range(4):
            k = 4 * j + b
            wait_gather(b)
            pltpu.sync_copy(bufs[b], acc.at[rowv.at[k]], add=True)
            gather(k + 4, b)
        return carry

    lax.fori_loop(0, NCHUNK // 4, quad0, 0)
    wait_gather(0)
    pltpu.sync_copy(buf0, acc.at[rowv.at[NCHUNK - 1]], add=True)
    gather(jnp.int32(NCHUNK + 3), 0)

    plsc.subcore_barrier()
    copy_out(0)
    plsc.subcore_barrier()
    zero_acc()
    plsc.subcore_barrier()

    def quad1(j, carry):
        for b in range(4):
            s = (b + 1) % 4
            k = NCHUNK + 4 * j + b
            wait_gather(s)
            pltpu.sync_copy(bufs[s], acc.at[rowv.at[k]], add=True)
            gather(k + 4, s)
        return carry

    lax.fori_loop(0, NCHUNK // 4, quad1, 0)
    s_tail = (NSTREAM - 1) % 4
    wait_gather(s_tail)
    pltpu.sync_copy(bufs[s_tail], acc.at[rowv.at[NSTREAM - 1]], add=True)
    for b in range(4):
        if b != s_tail:
            wait_gather(b)

    plsc.subcore_barrier()
    copy_out(1)


def kernel(x, edge_index, W1, W2):
    row = edge_index[0].astype(jnp.int32)
    col = edge_index[1].astype(jnp.int32)
    e_idx = jnp.arange(N_EDGES, dtype=jnp.int32)
    grp_of_edge = (e_idx // EPT) % TRASH_GROUPS
    trash = (WIN + grp_of_edge * TRASH_PER_GROUP
             + (e_idx % TRASH_PER_GROUP))
    row_lo = jnp.where(row < WIN, row, trash)
    row_hi = jnp.where(row >= WIN, row - WIN, trash)
    col3 = col.reshape(NW, NCHUNK, CHUNK)
    rowcat = jnp.concatenate(
        [row_lo.reshape(NW, NCHUNK, CHUNK),
         row_hi.reshape(NW, NCHUNK, CHUNK)], axis=1)

    feats = pl.pallas_call(
        _mm_body,
        grid=(N_BLKS,),
        in_specs=[pl.BlockSpec((ROW_BLK, D), lambda i: (i, 0)),
                  pl.BlockSpec((D, D), lambda i: (0, 0))],
        out_specs=pl.BlockSpec((ROW_BLK, D), lambda i: (i, 0)),
        out_shape=jax.ShapeDtypeStruct((N_NODES, D), jnp.float32),
    )(x, W2.T)

    zeros = jnp.zeros((CHUNK, D), jnp.float32)
    partials = _aggregate(feats, col3, rowcat, zeros)

    out = pl.pallas_call(
        _combine_body,
        grid=(N_BLKS,),
        in_specs=[pl.BlockSpec((ROW_BLK, D), lambda i: (i, 0)),
                  pl.BlockSpec((D, D), lambda i: (0, 0)),
                  pl.BlockSpec((2, ROW_BLK, D), lambda i: (0, i, 0))],
        out_specs=pl.BlockSpec((ROW_BLK, D), lambda i: (i, 0)),
        out_shape=jax.ShapeDtypeStruct((N_NODES, D), jnp.float32),
    )(x, W1.T, partials)
    return out
